# baseline (speedup 1.0000x reference)
import jax
import jax.numpy as jnp
from jax import lax
from jax.experimental import pallas as pl
from jax.experimental.pallas import tpu as pltpu
from jax.experimental.pallas import tpu_sc as plsc

D_MODEL = 64
NC, NS = 2, 16
NW = NC * NS
BB = 128


def _tec_body(x_hbm, tab_hbm, out_hbm,
              x_vA, x_vB, idx_vA, idx_vB, rR_vA, rA_vA, rR_vB, rA_vB,
              out_vA, out_vB, xsemA, xsemB, gsemA, gsemB, osemA, osemB):
    w = lax.axis_index("s") * NC + lax.axis_index("c")
    n_t = x_hbm.shape[0]
    b0 = w * BB

    def x_copies(t, x_v, xsem):
        return [
            pltpu.make_async_copy(x_hbm.at[t, 0, pl.ds(b0, BB)], x_v.at[0], xsem),
            pltpu.make_async_copy(x_hbm.at[t, 1, pl.ds(b0, BB)], x_v.at[1], xsem),
        ]

    def gather_copies(idx_v, rR_v, rA_v, gsem):
        return [
            pltpu.make_async_copy(tab_hbm.at[idx_v.at[0]], rR_v, gsem),
            pltpu.make_async_copy(tab_hbm.at[idx_v.at[1]], rA_v, gsem),
        ]

    def out_copies(t, out_v, osem):
        return [
            pltpu.make_async_copy(out_v.at[dblk], out_hbm.at[t, dblk, w], osem)
            for dblk in range(D_MODEL // 8)
        ]

    def start(copies):
        for c in copies:
            c.start()

    def wait(copies):
        for c in copies:
            c.wait()

    def compute_idx(x_v, idx_v):
        for r in range(2):
            for i in range(BB // 16):
                xv = x_v[r, pl.ds(i * 16, 16)]
                idx_v[r, pl.ds(i * 16, 16)] = (xv * 5000.0 + 5000.0).astype(jnp.int32)

    def transpose_add(rR_v, rA_v, out_v):
        @pl.loop(0, BB // 16)
        def _j(j):
            row = lax.iota(jnp.int32, 16) + j * 16
            for dblk in range(D_MODEL // 8):
                for din in range(8):
                    col = jnp.full((16,), dblk * 8 + din, jnp.int32)
                    a = plsc.load_gather(rR_v, [row, col])
                    b = plsc.load_gather(rA_v, [row, col])
                    out_v[dblk, din, pl.ds(j * 16, 16)] = a + b

    start(x_copies(0, x_vA, xsemA))
    start(x_copies(1, x_vB, xsemB))
    wait(x_copies(0, x_vA, xsemA))
    compute_idx(x_vA, idx_vA)
    start(gather_copies(idx_vA, rR_vA, rA_vA, gsemA))

    @pl.loop(0, n_t // 2)
    def _iter(k):
        a = 2 * k
        wait(x_copies(a + 1, x_vB, xsemB))
        compute_idx(x_vB, idx_vB)
        start(gather_copies(idx_vB, rR_vB, rA_vB, gsemB))

        @pl.when(a + 2 < n_t)
        def _():
            start(x_copies(a + 2, x_vA, xsemA))

        wait(gather_copies(idx_vA, rR_vA, rA_vA, gsemA))

        @pl.when(k >= 1)
        def _():
            wait(out_copies(a - 2, out_vA, osemA))

        transpose_add(rR_vA, rA_vA, out_vA)
        start(out_copies(a, out_vA, osemA))

        @pl.when(a + 2 < n_t)
        def _():
            wait(x_copies(a + 2, x_vA, xsemA))
            compute_idx(x_vA, idx_vA)
            start(gather_copies(idx_vA, rR_vA, rA_vA, gsemA))
            start(x_copies(a + 3, x_vB, xsemB))

        wait(gather_copies(idx_vB, rR_vB, rA_vB, gsemB))

        @pl.when(k >= 1)
        def _():
            wait(out_copies(a - 1, out_vB, osemB))

        transpose_add(rR_vB, rA_vB, out_vB)
        start(out_copies(a + 1, out_vB, osemB))

    wait(out_copies(n_t - 2, out_vA, osemA))
    wait(out_copies(n_t - 1, out_vB, osemB))


def kernel(x, pos_enc):
    b, t, _ = x.shape
    xt = jnp.transpose(x, (1, 2, 0))

    mesh = plsc.VectorSubcoreMesh(
        core_axis_name="c", subcore_axis_name="s", num_cores=NC, num_subcores=NS
    )
    run = pl.kernel(
        _tec_body,
        out_type=jax.ShapeDtypeStruct((t, D_MODEL // 8, b // BB, 8, BB), jnp.float32),
        mesh=mesh,
        scratch_types=[
            pltpu.VMEM((2, BB), jnp.float32),
            pltpu.VMEM((2, BB), jnp.float32),
            pltpu.VMEM((2, BB), jnp.int32),
            pltpu.VMEM((2, BB), jnp.int32),
            pltpu.VMEM((BB, D_MODEL), jnp.float32),
            pltpu.VMEM((BB, D_MODEL), jnp.float32),
            pltpu.VMEM((BB, D_MODEL), jnp.float32),
            pltpu.VMEM((BB, D_MODEL), jnp.float32),
            pltpu.VMEM((D_MODEL // 8, 8, BB), jnp.float32),
            pltpu.VMEM((D_MODEL // 8, 8, BB), jnp.float32),
            pltpu.SemaphoreType.DMA,
            pltpu.SemaphoreType.DMA,
            pltpu.SemaphoreType.DMA,
            pltpu.SemaphoreType.DMA,
            pltpu.SemaphoreType.DMA,
            pltpu.SemaphoreType.DMA,
        ],
        compiler_params=pltpu.CompilerParams(
            use_tc_tiling_on_sc=False, needs_layout_passes=False
        ),
    )
    out5 = run(xt, pos_enc)
    o = jnp.transpose(out5, (0, 1, 3, 2, 4))
    o = o.reshape(t, D_MODEL, b)
    return jnp.transpose(o, (2, 0, 1))


# speedup vs baseline: 1.9186x; 1.9186x over previous
import jax
import jax.numpy as jnp
from jax import lax
from jax.experimental import pallas as pl
from jax.experimental.pallas import tpu as pltpu
from jax.experimental.pallas import tpu_sc as plsc

D_MODEL = 64
NC, NS = 2, 16
NW = NC * NS
BB = 128


def _tec_body(x_hbm, tab_hbm, out_hbm,
              x_vA, x_vB, idx_vA, idx_vB, rR_vA, rA_vA, rR_vB, rA_vB,
              out_vA, out_vB, xsemA, xsemB, gsemA, gsemB, osemA, osemB):
    w = lax.axis_index("s") * NC + lax.axis_index("c")
    n_t = x_hbm.shape[0]
    b0 = w * BB

    def x_copies(t, x_v, xsem):
        return [
            pltpu.make_async_copy(x_hbm.at[t, 0, pl.ds(b0, BB)], x_v.at[0], xsem),
            pltpu.make_async_copy(x_hbm.at[t, 1, pl.ds(b0, BB)], x_v.at[1], xsem),
        ]

    def gather_copies(idx_v, rR_v, rA_v, gsem):
        return [
            pltpu.make_async_copy(tab_hbm.at[idx_v.at[0]], rR_v, gsem),
            pltpu.make_async_copy(tab_hbm.at[idx_v.at[1]], rA_v, gsem),
        ]

    def out_copies(t, out_v, osem):
        return [
            pltpu.make_async_copy(
                out_v.at[pl.ds(dblk * 8, 8)], out_hbm.at[t, dblk, w], osem)
            for dblk in range(D_MODEL // 8)
        ]

    def start(copies):
        for c in copies:
            c.start()

    def wait(copies):
        for c in copies:
            c.wait()

    def compute_idx(x_v, idx_v):
        for r in range(2):
            for i in range(BB // 16):
                xv = x_v[r, pl.ds(i * 16, 16)]
                idx_v[r, pl.ds(i * 16, 16)] = (xv * 5000.0 + 5000.0).astype(jnp.int32)

    def transpose_add(rR_v, rA_v, out_v):
        @pl.loop(0, BB, unroll=8)
        def _p(p):
            col = jnp.full((16,), p, jnp.int32)
            for d0 in range(0, D_MODEL, 16):
                row = lax.iota(jnp.int32, 16) + d0
                s = rR_v[p, pl.ds(d0, 16)] + rA_v[p, pl.ds(d0, 16)]
                plsc.store_scatter(out_v, [row, col], s)

    start(x_copies(0, x_vA, xsemA))
    start(x_copies(1, x_vB, xsemB))
    wait(x_copies(0, x_vA, xsemA))
    compute_idx(x_vA, idx_vA)
    start(gather_copies(idx_vA, rR_vA, rA_vA, gsemA))

    @pl.loop(0, n_t // 2)
    def _iter(k):
        a = 2 * k
        wait(x_copies(a + 1, x_vB, xsemB))
        compute_idx(x_vB, idx_vB)
        start(gather_copies(idx_vB, rR_vB, rA_vB, gsemB))

        @pl.when(a + 2 < n_t)
        def _():
            start(x_copies(a + 2, x_vA, xsemA))

        wait(gather_copies(idx_vA, rR_vA, rA_vA, gsemA))

        @pl.when(k >= 1)
        def _():
            wait(out_copies(a - 2, out_vA, osemA))

        transpose_add(rR_vA, rA_vA, out_vA)
        start(out_copies(a, out_vA, osemA))

        @pl.when(a + 2 < n_t)
        def _():
            wait(x_copies(a + 2, x_vA, xsemA))
            compute_idx(x_vA, idx_vA)
            start(gather_copies(idx_vA, rR_vA, rA_vA, gsemA))
            start(x_copies(a + 3, x_vB, xsemB))

        wait(gather_copies(idx_vB, rR_vB, rA_vB, gsemB))

        @pl.when(k >= 1)
        def _():
            wait(out_copies(a - 1, out_vB, osemB))

        transpose_add(rR_vB, rA_vB, out_vB)
        start(out_copies(a + 1, out_vB, osemB))

    wait(out_copies(n_t - 2, out_vA, osemA))
    wait(out_copies(n_t - 1, out_vB, osemB))


def kernel(x, pos_enc):
    b, t, _ = x.shape
    xt = jnp.transpose(x, (1, 2, 0))

    mesh = plsc.VectorSubcoreMesh(
        core_axis_name="c", subcore_axis_name="s", num_cores=NC, num_subcores=NS
    )
    run = pl.kernel(
        _tec_body,
        out_type=jax.ShapeDtypeStruct((t, D_MODEL // 8, b // BB, 8, BB), jnp.float32),
        mesh=mesh,
        scratch_types=[
            pltpu.VMEM((2, BB), jnp.float32),
            pltpu.VMEM((2, BB), jnp.float32),
            pltpu.VMEM((2, BB), jnp.int32),
            pltpu.VMEM((2, BB), jnp.int32),
            pltpu.VMEM((BB, D_MODEL), jnp.float32),
            pltpu.VMEM((BB, D_MODEL), jnp.float32),
            pltpu.VMEM((BB, D_MODEL), jnp.float32),
            pltpu.VMEM((BB, D_MODEL), jnp.float32),
            pltpu.VMEM((D_MODEL, BB), jnp.float32),
            pltpu.VMEM((D_MODEL, BB), jnp.float32),
            pltpu.SemaphoreType.DMA,
            pltpu.SemaphoreType.DMA,
            pltpu.SemaphoreType.DMA,
            pltpu.SemaphoreType.DMA,
            pltpu.SemaphoreType.DMA,
            pltpu.SemaphoreType.DMA,
        ],
        compiler_params=pltpu.CompilerParams(
            use_tc_tiling_on_sc=False, needs_layout_passes=False
        ),
    )
    out5 = run(xt, pos_enc)
    o = jnp.transpose(out5, (0, 1, 3, 2, 4))
    o = o.reshape(t, D_MODEL, b)
    return jnp.transpose(o, (2, 0, 1))
